# baseline (speedup 1.0000x reference)
import functools

import jax
import jax.numpy as jnp
from jax import lax
from jax.experimental import pallas as pl
from jax.experimental.pallas import tpu as pltpu
from jax.experimental.pallas import tpu_sc as plsc

VOCAB = 1000000
D = 64
N = 4096 * 200

NC = 2
NS = 16
NW = NC * NS
ROWS_PER_W = N // NW
SUB = 128
CHUNK = 512
N_SUB = CHUNK // SUB
N_CHUNKS = ROWS_PER_W // CHUNK
N_PAIRS = N_CHUNKS // 2

_LN2 = 0.6931471805599453

_sc_mesh = plsc.VectorSubcoreMesh(core_axis_name="c", subcore_axis_name="s")


def _ln(v):
    bits = plsc.bitcast(v, jnp.int32)
    e = ((bits >> 23) & 0xFF) - 127
    m = plsc.bitcast((bits & 0x007FFFFF) | 0x3F800000, jnp.float32)
    z = (m - 1.0) / (m + 1.0)
    z2 = z * z
    p = 1.0 / 7.0 + z2 * (1.0 / 9.0)
    p = 1.0 / 5.0 + z2 * p
    p = 1.0 / 3.0 + z2 * p
    lnm = 2.0 * z * (1.0 + z2 * p)
    return lnm + e.astype(jnp.float32) * _LN2


@functools.partial(
    pl.kernel,
    mesh=_sc_mesh,
    out_type=(
        jax.ShapeDtypeStruct((N, D), jnp.float32),
        jax.ShapeDtypeStruct((NW, 16), jnp.float32),
    ),
    scratch_types=[
        [pltpu.VMEM((N_SUB, SUB), jnp.int32) for _ in range(2)],
        [pltpu.VMEM((CHUNK,), jnp.int32) for _ in range(2)],
        [pltpu.VMEM((CHUNK, D), jnp.float32) for _ in range(2)],
        pltpu.VMEM((16,), jnp.float32),
        [pltpu.SemaphoreType.DMA for _ in range(2)],
        [pltpu.SemaphoreType.DMA for _ in range(2)],
    ],
    compiler_params=pltpu.CompilerParams(
        use_tc_tiling_on_sc=False, needs_layout_passes=False),
)
def _sc_embed_ce(idx_hbm, tgt_hbm, table_hbm, out_hbm, part_hbm,
                 idx_v, tgt_v, bufs, accv, sems, semw):
    wid = lax.axis_index("s") * NC + lax.axis_index("c")
    grp0 = wid * (ROWS_PER_W // SUB)
    row0 = wid * ROWS_PER_W
    accv[...] = jnp.zeros((16,), jnp.float32)
    lane = lax.iota(jnp.int32, 16)

    def stage(c, slot):
        g = grp0 + c * N_SUB
        pltpu.sync_copy(idx_hbm.at[pl.ds(g, N_SUB)], idx_v[slot])
        pltpu.sync_copy(tgt_hbm.at[pl.ds(g * SUB, CHUNK)], tgt_v[slot])

    def fire(slot):
        return [
            pltpu.async_copy(
                table_hbm.at[idx_v[slot].at[j]],
                bufs[slot].at[pl.ds(j * SUB, SUB)],
                sems[slot],
            )
            for j in range(N_SUB)
        ]

    def ce(slot):
        buf = bufs[slot]
        tgt = tgt_v[slot]

        def grp_body(gi, carry2):
            r0 = gi * 16
            tgt16 = tgt[pl.ds(r0, 16)]
            s_vec = jnp.zeros((16,), jnp.float32)
            for r in range(16):
                e0 = jnp.exp(buf[r0 + r, pl.ds(0, 16)])
                e1 = jnp.exp(buf[r0 + r, pl.ds(16, 16)])
                e2 = jnp.exp(buf[r0 + r, pl.ds(32, 16)])
                e3 = jnp.exp(buf[r0 + r, pl.ds(48, 16)])
                s = jnp.sum((e0 + e1) + (e2 + e3))
                s_vec = jnp.where(lane == r, s, s_vec)
            picked = plsc.load_gather(buf, [r0 + lane, tgt16])
            accv[...] = accv[...] + (_ln(s_vec) - picked)
            return carry2

        lax.fori_loop(0, CHUNK // 16, grp_body, 0)

    def out_copy(c, slot):
        return pltpu.async_copy(
            bufs[slot],
            out_hbm.at[pl.ds(row0 + c * CHUNK, CHUNK)],
            semw[slot],
        )

    stage(0, 0)
    fire(0)

    def pair_body(p, carry):
        a = 2 * p
        stage(a + 1, 1)

        @pl.when(p > 0)
        def _():
            pltpu.make_async_copy(
                bufs[1],
                out_hbm.at[pl.ds(row0 + (a - 1) * CHUNK, CHUNK)],
                semw[1],
            ).wait()

        fire(1)
        for j in range(N_SUB):
            pltpu.make_async_copy(
                table_hbm.at[idx_v[0].at[j]],
                bufs[0].at[pl.ds(j * SUB, SUB)],
                sems[0],
            ).wait()
        ce(0)
        h0 = out_copy(a, 0)

        @pl.when(p < N_PAIRS - 1)
        def _():
            stage(a + 2, 0)
            h0.wait()
            fire(0)

        for j in range(N_SUB):
            pltpu.make_async_copy(
                table_hbm.at[idx_v[1].at[j]],
                bufs[1].at[pl.ds(j * SUB, SUB)],
                sems[1],
            ).wait()
        ce(1)
        out_copy(a + 1, 1)
        return carry

    lax.fori_loop(0, N_PAIRS, pair_body, 0)
    pltpu.make_async_copy(
        bufs[0],
        out_hbm.at[pl.ds(row0 + (N_CHUNKS - 2) * CHUNK, CHUNK)],
        semw[0],
    ).wait()
    pltpu.make_async_copy(
        bufs[1],
        out_hbm.at[pl.ds(row0 + (N_CHUNKS - 1) * CHUNK, CHUNK)],
        semw[1],
    ).wait()
    pltpu.sync_copy(accv, part_hbm.at[wid])


_PB = 512
_PG = -(-VOCAB // _PB)


def _tc_pad_body(wt_ref, out_ref):
    x = wt_ref[...]
    y = jnp.transpose(x, (1, 0))
    out_ref[...] = jnp.concatenate(
        [y, jnp.zeros((_PB, 128 - D), jnp.float32)], axis=1)


_tc_pad = pl.pallas_call(
    _tc_pad_body,
    grid=(_PG,),
    in_specs=[pl.BlockSpec((D, _PB), lambda i: (0, i))],
    out_specs=pl.BlockSpec((_PB, 128), lambda i: (i, 0)),
    out_shape=jax.ShapeDtypeStruct((VOCAB, 128), jnp.float32),
)


def kernel(inputs, targets, wte):
    idx2 = (inputs.astype(jnp.int32).reshape(-1) * 2).reshape(N // SUB, SUB)
    tgt = targets.astype(jnp.int32).reshape(N)
    table = _tc_pad(wte.T).reshape(2 * VOCAB, D)
    logits2, partials = _sc_embed_ce(idx2, tgt, table)
    loss = jnp.sum(partials) * (1.0 / N)
    return (logits2, loss)


# speedup vs baseline: 1.8058x; 1.8058x over previous
import functools

import jax
import jax.numpy as jnp
from jax import lax
from jax.experimental import pallas as pl
from jax.experimental.pallas import tpu as pltpu
from jax.experimental.pallas import tpu_sc as plsc

VOCAB = 1000000
D = 64
N = 4096 * 200

NC = 2
NS = 16
NW = NC * NS
ROWS_PER_W = N // NW
SUB = 128
CHUNK = 512
N_SUB = CHUNK // SUB
N_CHUNKS = ROWS_PER_W // CHUNK
N_PAIRS = N_CHUNKS // 2

_LN2 = 0.6931471805599453

_sc_mesh = plsc.VectorSubcoreMesh(core_axis_name="c", subcore_axis_name="s")


def _ln(v):
    bits = plsc.bitcast(v, jnp.int32)
    e = ((bits >> 23) & 0xFF) - 127
    m = plsc.bitcast((bits & 0x007FFFFF) | 0x3F800000, jnp.float32)
    z = (m - 1.0) / (m + 1.0)
    z2 = z * z
    p = 1.0 / 7.0 + z2 * (1.0 / 9.0)
    p = 1.0 / 5.0 + z2 * p
    p = 1.0 / 3.0 + z2 * p
    lnm = 2.0 * z * (1.0 + z2 * p)
    return lnm + e.astype(jnp.float32) * _LN2


@functools.partial(
    pl.kernel,
    mesh=_sc_mesh,
    out_type=(
        jax.ShapeDtypeStruct((N, D), jnp.float32),
        jax.ShapeDtypeStruct((NW, 16), jnp.float32),
    ),
    scratch_types=[
        [pltpu.VMEM((N_SUB, SUB), jnp.int32) for _ in range(2)],
        [pltpu.VMEM((CHUNK,), jnp.int32) for _ in range(2)],
        [pltpu.VMEM((CHUNK, D), jnp.float32) for _ in range(2)],
        pltpu.VMEM((16,), jnp.float32),
        [pltpu.SemaphoreType.DMA for _ in range(2)],
        [pltpu.SemaphoreType.DMA for _ in range(2)],
    ],
    compiler_params=pltpu.CompilerParams(
        use_tc_tiling_on_sc=False, needs_layout_passes=False),
)
def _sc_embed_ce(idx_hbm, tgt_hbm, table_hbm, out_hbm, part_hbm,
                 idx_v, tgt_v, bufs, accv, sems, semw):
    wid = lax.axis_index("s") * NC + lax.axis_index("c")
    grp0 = wid * (ROWS_PER_W // SUB)
    row0 = wid * ROWS_PER_W
    accv[...] = jnp.zeros((16,), jnp.float32)
    lane = lax.iota(jnp.int32, 16)

    def stage(c, slot):
        g = grp0 + c * N_SUB
        pltpu.sync_copy(idx_hbm.at[pl.ds(g, N_SUB)], idx_v[slot])
        pltpu.sync_copy(tgt_hbm.at[pl.ds(g * SUB, CHUNK)], tgt_v[slot])

    def fire(slot):
        return [
            pltpu.async_copy(
                table_hbm.at[idx_v[slot].at[j]],
                bufs[slot].at[pl.ds(j * SUB, SUB)],
                sems[slot],
            )
            for j in range(N_SUB)
        ]

    def ce(slot):
        buf = bufs[slot]
        tgt = tgt_v[slot]

        def grp_body(gi, carry2):
            r0 = gi * 16
            tgt16 = tgt[pl.ds(r0, 16)]
            s_vec = jnp.zeros((16,), jnp.float32)
            for r in range(16):
                e0 = jnp.exp(buf[r0 + r, pl.ds(0, 16)])
                e1 = jnp.exp(buf[r0 + r, pl.ds(16, 16)])
                e2 = jnp.exp(buf[r0 + r, pl.ds(32, 16)])
                e3 = jnp.exp(buf[r0 + r, pl.ds(48, 16)])
                s = jnp.sum((e0 + e1) + (e2 + e3))
                s_vec = jnp.where(lane == r, s, s_vec)
            picked = plsc.load_gather(buf, [r0 + lane, tgt16])
            accv[...] = accv[...] + (_ln(s_vec) - picked)
            return carry2

        lax.fori_loop(0, CHUNK // 16, grp_body, 0)

    def out_copy(c, slot):
        return pltpu.async_copy(
            bufs[slot],
            out_hbm.at[pl.ds(row0 + c * CHUNK, CHUNK)],
            semw[slot],
        )

    stage(0, 0)
    fire(0)

    def pair_body(p, carry):
        a = 2 * p
        stage(a + 1, 1)

        @pl.when(p > 0)
        def _():
            pltpu.make_async_copy(
                bufs[1],
                out_hbm.at[pl.ds(row0 + (a - 1) * CHUNK, CHUNK)],
                semw[1],
            ).wait()

        fire(1)
        for j in range(N_SUB):
            pltpu.make_async_copy(
                table_hbm.at[idx_v[0].at[j]],
                bufs[0].at[pl.ds(j * SUB, SUB)],
                sems[0],
            ).wait()
        ce(0)
        h0 = out_copy(a, 0)

        @pl.when(p < N_PAIRS - 1)
        def _():
            stage(a + 2, 0)
            h0.wait()
            fire(0)

        for j in range(N_SUB):
            pltpu.make_async_copy(
                table_hbm.at[idx_v[1].at[j]],
                bufs[1].at[pl.ds(j * SUB, SUB)],
                sems[1],
            ).wait()
        ce(1)
        out_copy(a + 1, 1)
        return carry

    lax.fori_loop(0, N_PAIRS, pair_body, 0)
    pltpu.make_async_copy(
        bufs[0],
        out_hbm.at[pl.ds(row0 + (N_CHUNKS - 2) * CHUNK, CHUNK)],
        semw[0],
    ).wait()
    pltpu.make_async_copy(
        bufs[1],
        out_hbm.at[pl.ds(row0 + (N_CHUNKS - 1) * CHUNK, CHUNK)],
        semw[1],
    ).wait()
    pltpu.sync_copy(accv, part_hbm.at[wid])


_PB = 4096
_PG = -(-VOCAB // _PB)


def _tc_pad_body(wt_ref, out_ref):
    x = wt_ref[...]
    y = jnp.transpose(x, (1, 0))
    out_ref[:, pl.ds(0, D)] = y


_tc_pad = pl.pallas_call(
    _tc_pad_body,
    grid=(_PG,),
    in_specs=[pl.BlockSpec((D, _PB), lambda i: (0, i))],
    out_specs=pl.BlockSpec((_PB, 128), lambda i: (i, 0)),
    out_shape=jax.ShapeDtypeStruct((VOCAB, 128), jnp.float32),
)


def kernel(inputs, targets, wte):
    idx2 = (inputs.astype(jnp.int32).reshape(-1) * 2).reshape(N // SUB, SUB)
    tgt = targets.astype(jnp.int32).reshape(N)
    table = _tc_pad(wte.T).reshape(2 * VOCAB, D)
    logits2, partials = _sc_embed_ce(idx2, tgt, table)
    loss = jnp.sum(partials) * (1.0 / N)
    return (logits2, loss)


# speedup vs baseline: 2.5338x; 1.4031x over previous
import functools

import jax
import jax.numpy as jnp
from jax import lax
from jax.experimental import pallas as pl
from jax.experimental.pallas import tpu as pltpu
from jax.experimental.pallas import tpu_sc as plsc

VOCAB = 1000000
D = 64
N = 4096 * 200

NC = 2
NS = 16
NW = NC * NS
ROWS_PER_W = N // NW
SUB = 128
CHUNK = 512
N_SUB = CHUNK // SUB
N_CHUNKS = ROWS_PER_W // CHUNK
N_PAIRS = N_CHUNKS // 2

_LN2 = 0.6931471805599453

_sc_mesh = plsc.VectorSubcoreMesh(core_axis_name="c", subcore_axis_name="s")


def _ln(v):
    bits = plsc.bitcast(v, jnp.int32)
    e = ((bits >> 23) & 0xFF) - 127
    m = plsc.bitcast((bits & 0x007FFFFF) | 0x3F800000, jnp.float32)
    z = (m - 1.0) / (m + 1.0)
    z2 = z * z
    p = 1.0 / 7.0 + z2 * (1.0 / 9.0)
    p = 1.0 / 5.0 + z2 * p
    p = 1.0 / 3.0 + z2 * p
    lnm = 2.0 * z * (1.0 + z2 * p)
    return lnm + e.astype(jnp.float32) * _LN2


@functools.partial(
    pl.kernel,
    mesh=_sc_mesh,
    out_type=(
        jax.ShapeDtypeStruct((N, 128), jnp.float32),
        jax.ShapeDtypeStruct((NW, 16), jnp.float32),
    ),
    scratch_types=[
        [pltpu.VMEM((N_SUB, SUB), jnp.int32) for _ in range(2)],
        [pltpu.VMEM((CHUNK,), jnp.int32) for _ in range(2)],
        [pltpu.VMEM((CHUNK, D), jnp.float32) for _ in range(2)],
        pltpu.VMEM((16,), jnp.float32),
        [pltpu.SemaphoreType.DMA for _ in range(2)],
        [pltpu.SemaphoreType.DMA for _ in range(2)],
    ],
    compiler_params=pltpu.CompilerParams(
        use_tc_tiling_on_sc=False, needs_layout_passes=False),
)
def _sc_embed_ce(idx_hbm, tgt_hbm, table_hbm, out_hbm, part_hbm,
                 idx_v, tgt_v, bufs, accv, sems, semw):
    wid = lax.axis_index("s") * NC + lax.axis_index("c")
    grp0 = wid * (ROWS_PER_W // SUB)
    row0 = wid * ROWS_PER_W
    accv[...] = jnp.zeros((16,), jnp.float32)
    lane = lax.iota(jnp.int32, 16)

    def stage(c, slot):
        g = grp0 + c * N_SUB
        pltpu.sync_copy(idx_hbm.at[pl.ds(g, N_SUB)], idx_v[slot])
        pltpu.sync_copy(tgt_hbm.at[pl.ds(g * SUB, CHUNK)], tgt_v[slot])

    def fire(slot):
        return [
            pltpu.async_copy(
                table_hbm.at[idx_v[slot].at[j]],
                bufs[slot].at[pl.ds(j * SUB, SUB)],
                sems[slot],
            )
            for j in range(N_SUB)
        ]

    def ce(slot):
        buf = bufs[slot]
        tgt = tgt_v[slot]

        def grp_body(gi, carry2):
            r0 = gi * 16
            tgt16 = tgt[pl.ds(r0, 16)]
            s_vec = jnp.zeros((16,), jnp.float32)
            for r in range(16):
                e0 = jnp.exp(buf[r0 + r, pl.ds(0, 16)])
                e1 = jnp.exp(buf[r0 + r, pl.ds(16, 16)])
                e2 = jnp.exp(buf[r0 + r, pl.ds(32, 16)])
                e3 = jnp.exp(buf[r0 + r, pl.ds(48, 16)])
                s = jnp.sum((e0 + e1) + (e2 + e3))
                s_vec = jnp.where(lane == r, s, s_vec)
            picked = plsc.load_gather(buf, [r0 + lane, tgt16])
            accv[...] = accv[...] + (_ln(s_vec) - picked)
            return carry2

        lax.fori_loop(0, CHUNK // 16, grp_body, 0)

    def out_copy(c, slot):
        return pltpu.async_copy(
            bufs[slot],
            out_hbm.at[pl.ds(row0 + c * CHUNK, CHUNK), pl.ds(0, D)],
            semw[slot],
        )

    stage(0, 0)
    fire(0)

    def pair_body(p, carry):
        a = 2 * p
        stage(a + 1, 1)

        @pl.when(p > 0)
        def _():
            pltpu.make_async_copy(
                bufs[1],
                out_hbm.at[pl.ds(row0 + (a - 1) * CHUNK, CHUNK), pl.ds(0, D)],
                semw[1],
            ).wait()

        fire(1)
        for j in range(N_SUB):
            pltpu.make_async_copy(
                table_hbm.at[idx_v[0].at[j]],
                bufs[0].at[pl.ds(j * SUB, SUB)],
                sems[0],
            ).wait()
        ce(0)
        h0 = out_copy(a, 0)

        @pl.when(p < N_PAIRS - 1)
        def _():
            stage(a + 2, 0)
            h0.wait()
            fire(0)

        for j in range(N_SUB):
            pltpu.make_async_copy(
                table_hbm.at[idx_v[1].at[j]],
                bufs[1].at[pl.ds(j * SUB, SUB)],
                sems[1],
            ).wait()
        ce(1)
        out_copy(a + 1, 1)
        return carry

    lax.fori_loop(0, N_PAIRS, pair_body, 0)
    pltpu.make_async_copy(
        bufs[0],
        out_hbm.at[pl.ds(row0 + (N_CHUNKS - 2) * CHUNK, CHUNK), pl.ds(0, D)],
        semw[0],
    ).wait()
    pltpu.make_async_copy(
        bufs[1],
        out_hbm.at[pl.ds(row0 + (N_CHUNKS - 1) * CHUNK, CHUNK), pl.ds(0, D)],
        semw[1],
    ).wait()
    pltpu.sync_copy(accv, part_hbm.at[wid])


_PB = 4096
_PG = -(-VOCAB // _PB)


def _tc_pad_body(wt_ref, out_ref):
    x = wt_ref[...]
    y = jnp.transpose(x, (1, 0))
    out_ref[:, pl.ds(0, D)] = y


_tc_pad = pl.pallas_call(
    _tc_pad_body,
    grid=(_PG,),
    in_specs=[pl.BlockSpec((D, _PB), lambda i: (0, i))],
    out_specs=pl.BlockSpec((_PB, 128), lambda i: (i, 0)),
    out_shape=jax.ShapeDtypeStruct((VOCAB, 128), jnp.float32),
)


def kernel(inputs, targets, wte):
    idx2 = (inputs.astype(jnp.int32).reshape(-1) * 2).reshape(N // SUB, SUB)
    tgt = targets.astype(jnp.int32).reshape(N)
    table = _tc_pad(wte.T).reshape(2 * VOCAB, D)
    logits_pad, partials = _sc_embed_ce(idx2, tgt, table)
    logits2 = logits_pad[:, :D]
    loss = jnp.sum(partials) * (1.0 / N)
    return (logits2, loss)


# speedup vs baseline: 2.7980x; 1.1042x over previous
import functools

import jax
import jax.numpy as jnp
from jax import lax
from jax.experimental import pallas as pl
from jax.experimental.pallas import tpu as pltpu
from jax.experimental.pallas import tpu_sc as plsc

VOCAB = 1000000
D = 64
N = 4096 * 200

NC = 2
NS = 16
NW = NC * NS
ROWS_PER_W = N // NW
SUB = 128
CHUNK = 512
N_SUB = CHUNK // SUB
N_CHUNKS = ROWS_PER_W // CHUNK
N_PAIRS = N_CHUNKS // 2

_LN2 = 0.6931471805599453

_sc_mesh = plsc.VectorSubcoreMesh(core_axis_name="c", subcore_axis_name="s")


def _ln(v):
    bits = plsc.bitcast(v, jnp.int32)
    e = ((bits >> 23) & 0xFF) - 127
    m = plsc.bitcast((bits & 0x007FFFFF) | 0x3F800000, jnp.float32)
    z = (m - 1.0) / (m + 1.0)
    z2 = z * z
    p = 1.0 / 7.0 + z2 * (1.0 / 9.0)
    p = 1.0 / 5.0 + z2 * p
    p = 1.0 / 3.0 + z2 * p
    lnm = 2.0 * z * (1.0 + z2 * p)
    return lnm + e.astype(jnp.float32) * _LN2


@functools.partial(
    pl.kernel,
    mesh=_sc_mesh,
    out_type=(
        jax.ShapeDtypeStruct((N, 128), jnp.float32),
        jax.ShapeDtypeStruct((NW, 16), jnp.float32),
    ),
    scratch_types=[
        [pltpu.VMEM((N_SUB, SUB), jnp.int32) for _ in range(2)],
        [pltpu.VMEM((CHUNK,), jnp.int32) for _ in range(2)],
        [pltpu.VMEM((CHUNK, D), jnp.float32) for _ in range(2)],
        pltpu.VMEM((16,), jnp.float32),
        [pltpu.SemaphoreType.DMA for _ in range(2)],
        [pltpu.SemaphoreType.DMA for _ in range(2)],
    ],
    compiler_params=pltpu.CompilerParams(
        use_tc_tiling_on_sc=False, needs_layout_passes=False),
)
def _sc_embed_ce(idx_hbm, tgt_hbm, table_hbm, out_hbm, part_hbm,
                 idx_v, tgt_v, bufs, accv, sems, semw):
    wid = lax.axis_index("s") * NC + lax.axis_index("c")
    grp0 = wid * (ROWS_PER_W // SUB)
    row0 = wid * ROWS_PER_W
    accv[...] = jnp.zeros((16,), jnp.float32)
    lane = lax.iota(jnp.int32, 16)

    def stage(c, slot):
        g = grp0 + c * N_SUB
        pltpu.sync_copy(idx_hbm.at[pl.ds(g, N_SUB)], idx_v[slot])
        pltpu.sync_copy(tgt_hbm.at[pl.ds(g * SUB, CHUNK)], tgt_v[slot])

    def fire(slot):
        return [
            pltpu.async_copy(
                table_hbm.at[idx_v[slot].at[j]],
                bufs[slot].at[pl.ds(j * SUB, SUB)],
                sems[slot],
            )
            for j in range(N_SUB)
        ]

    def ce(slot):
        buf = bufs[slot]
        tgt = tgt_v[slot]

        def grp_body(gi, carry2):
            r0 = gi * 16
            tgt16 = tgt[pl.ds(r0, 16)]
            s_vec = jnp.zeros((16,), jnp.float32)
            for r in range(16):
                e0 = jnp.exp(buf[r0 + r, pl.ds(0, 16)])
                e1 = jnp.exp(buf[r0 + r, pl.ds(16, 16)])
                e2 = jnp.exp(buf[r0 + r, pl.ds(32, 16)])
                e3 = jnp.exp(buf[r0 + r, pl.ds(48, 16)])
                s = jnp.sum((e0 + e1) + (e2 + e3))
                s_vec = jnp.where(lane == r, s, s_vec)
            picked = plsc.load_gather(buf, [r0 + lane, tgt16])
            accv[...] = accv[...] + (_ln(s_vec) - picked)
            return carry2

        lax.fori_loop(0, CHUNK // 16, grp_body, 0)

    def out_copy(c, slot):
        return pltpu.async_copy(
            bufs[slot],
            out_hbm.at[pl.ds(row0 + c * CHUNK, CHUNK), pl.ds(0, D)],
            semw[slot],
        )

    stage(0, 0)
    fire(0)

    def pair_body(p, carry):
        a = 2 * p
        stage(a + 1, 1)

        @pl.when(p > 0)
        def _():
            pltpu.make_async_copy(
                bufs[1],
                out_hbm.at[pl.ds(row0 + (a - 1) * CHUNK, CHUNK), pl.ds(0, D)],
                semw[1],
            ).wait()

        fire(1)
        for j in range(N_SUB):
            pltpu.make_async_copy(
                table_hbm.at[idx_v[0].at[j]],
                bufs[0].at[pl.ds(j * SUB, SUB)],
                sems[0],
            ).wait()
        ce(0)
        h0 = out_copy(a, 0)

        @pl.when(p < N_PAIRS - 1)
        def _():
            stage(a + 2, 0)
            h0.wait()
            fire(0)

        for j in range(N_SUB):
            pltpu.make_async_copy(
                table_hbm.at[idx_v[1].at[j]],
                bufs[1].at[pl.ds(j * SUB, SUB)],
                sems[1],
            ).wait()
        ce(1)
        out_copy(a + 1, 1)
        return carry

    lax.fori_loop(0, N_PAIRS, pair_body, 0)
    pltpu.make_async_copy(
        bufs[0],
        out_hbm.at[pl.ds(row0 + (N_CHUNKS - 2) * CHUNK, CHUNK), pl.ds(0, D)],
        semw[0],
    ).wait()
    pltpu.make_async_copy(
        bufs[1],
        out_hbm.at[pl.ds(row0 + (N_CHUNKS - 1) * CHUNK, CHUNK), pl.ds(0, D)],
        semw[1],
    ).wait()
    pltpu.sync_copy(accv, part_hbm.at[wid])


_PB = 8192
_PG = -(-VOCAB // _PB)


def _tc_pad_body(wt_ref, out_ref):
    x = wt_ref[...]
    y = jnp.transpose(x, (1, 0))
    out_ref[:, pl.ds(0, D)] = y


_tc_pad = pl.pallas_call(
    _tc_pad_body,
    grid=(_PG,),
    in_specs=[pl.BlockSpec((D, _PB), lambda i: (0, i))],
    out_specs=pl.BlockSpec((_PB, 128), lambda i: (i, 0)),
    out_shape=jax.ShapeDtypeStruct((VOCAB, 128), jnp.float32),
)


def kernel(inputs, targets, wte):
    idx2 = (inputs.astype(jnp.int32).reshape(-1) * 2).reshape(N // SUB, SUB)
    tgt = targets.astype(jnp.int32).reshape(N)
    table = _tc_pad(wte.T).reshape(2 * VOCAB, D)
    logits_pad, partials = _sc_embed_ce(idx2, tgt, table)
    logits2 = logits_pad[:, :D]
    loss = jnp.sum(partials) * (1.0 / N)
    return (logits2, loss)


# speedup vs baseline: 2.8813x; 1.0298x over previous
import functools

import jax
import jax.numpy as jnp
from jax import lax
from jax.experimental import pallas as pl
from jax.experimental.pallas import tpu as pltpu
from jax.experimental.pallas import tpu_sc as plsc

VOCAB = 1000000
D = 64
N = 4096 * 200

NC = 2
NS = 16
NW = NC * NS
ROWS_PER_W = N // NW
SUB = 128
CHUNK = 512
N_SUB = CHUNK // SUB
N_CHUNKS = ROWS_PER_W // CHUNK
N_PAIRS = N_CHUNKS // 2

_LN2 = 0.6931471805599453

_sc_mesh = plsc.VectorSubcoreMesh(core_axis_name="c", subcore_axis_name="s")


def _ln(v):
    bits = plsc.bitcast(v, jnp.int32)
    e = ((bits >> 23) & 0xFF) - 127
    m = plsc.bitcast((bits & 0x007FFFFF) | 0x3F800000, jnp.float32)
    z = (m - 1.0) / (m + 1.0)
    z2 = z * z
    p = 1.0 / 7.0 + z2 * (1.0 / 9.0)
    p = 1.0 / 5.0 + z2 * p
    p = 1.0 / 3.0 + z2 * p
    lnm = 2.0 * z * (1.0 + z2 * p)
    return lnm + e.astype(jnp.float32) * _LN2


@functools.partial(
    pl.kernel,
    mesh=_sc_mesh,
    out_type=(
        jax.ShapeDtypeStruct((N, 128), jnp.float32),
        jax.ShapeDtypeStruct((NW, 16), jnp.float32),
    ),
    scratch_types=[
        [pltpu.VMEM((N_SUB, SUB), jnp.int32) for _ in range(2)],
        [pltpu.VMEM((CHUNK,), jnp.int32) for _ in range(2)],
        [pltpu.VMEM((CHUNK, D), jnp.float32) for _ in range(2)],
        pltpu.VMEM((16,), jnp.float32),
        [pltpu.SemaphoreType.DMA for _ in range(2)],
        [pltpu.SemaphoreType.DMA for _ in range(2)],
    ],
    compiler_params=pltpu.CompilerParams(
        use_tc_tiling_on_sc=False, needs_layout_passes=False),
)
def _sc_embed_ce(idx_hbm, tgt_hbm, table_hbm, out_hbm, part_hbm,
                 idx_v, tgt_v, bufs, accv, sems, semw):
    wid = lax.axis_index("s") * NC + lax.axis_index("c")
    grp0 = wid * (ROWS_PER_W // SUB)
    row0 = wid * ROWS_PER_W
    accv[...] = jnp.zeros((16,), jnp.float32)
    lane = lax.iota(jnp.int32, 16)

    def stage(c, slot):
        g = grp0 + c * N_SUB
        pltpu.sync_copy(idx_hbm.at[pl.ds(g, N_SUB)], idx_v[slot])
        pltpu.sync_copy(tgt_hbm.at[pl.ds(g * SUB, CHUNK)], tgt_v[slot])

    def fire(slot):
        return [
            pltpu.async_copy(
                table_hbm.at[idx_v[slot].at[j]],
                bufs[slot].at[pl.ds(j * SUB, SUB)],
                sems[slot],
            )
            for j in range(N_SUB)
        ]

    def ce(slot):
        buf = bufs[slot]
        tgt = tgt_v[slot]

        def grp_body(gi, carry2):
            r0 = gi * 16
            tgt16 = tgt[pl.ds(r0, 16)]
            s_vec = jnp.zeros((16,), jnp.float32)
            for r in range(16):
                e0 = jnp.exp(buf[r0 + r, pl.ds(0, 16)])
                e1 = jnp.exp(buf[r0 + r, pl.ds(16, 16)])
                e2 = jnp.exp(buf[r0 + r, pl.ds(32, 16)])
                e3 = jnp.exp(buf[r0 + r, pl.ds(48, 16)])
                s = jnp.sum((e0 + e1) + (e2 + e3))
                s_vec = jnp.where(lane == r, s, s_vec)
            picked = plsc.load_gather(buf, [r0 + lane, tgt16])
            accv[...] = accv[...] + (_ln(s_vec) - picked)
            return carry2

        lax.fori_loop(0, CHUNK // 16, grp_body, 0)

    def out_copy(c, slot):
        return pltpu.async_copy(
            bufs[slot],
            out_hbm.at[pl.ds(row0 + c * CHUNK, CHUNK), pl.ds(0, D)],
            semw[slot],
        )

    stage(0, 0)
    fire(0)

    def pair_body(p, carry):
        a = 2 * p
        stage(a + 1, 1)

        @pl.when(p > 0)
        def _():
            pltpu.make_async_copy(
                bufs[1],
                out_hbm.at[pl.ds(row0 + (a - 1) * CHUNK, CHUNK), pl.ds(0, D)],
                semw[1],
            ).wait()

        fire(1)
        for j in range(N_SUB):
            pltpu.make_async_copy(
                table_hbm.at[idx_v[0].at[j]],
                bufs[0].at[pl.ds(j * SUB, SUB)],
                sems[0],
            ).wait()
        ce(0)
        h0 = out_copy(a, 0)

        @pl.when(p < N_PAIRS - 1)
        def _():
            stage(a + 2, 0)
            h0.wait()
            fire(0)

        for j in range(N_SUB):
            pltpu.make_async_copy(
                table_hbm.at[idx_v[1].at[j]],
                bufs[1].at[pl.ds(j * SUB, SUB)],
                sems[1],
            ).wait()
        ce(1)
        out_copy(a + 1, 1)
        return carry

    lax.fori_loop(0, N_PAIRS, pair_body, 0)
    pltpu.make_async_copy(
        bufs[0],
        out_hbm.at[pl.ds(row0 + (N_CHUNKS - 2) * CHUNK, CHUNK), pl.ds(0, D)],
        semw[0],
    ).wait()
    pltpu.make_async_copy(
        bufs[1],
        out_hbm.at[pl.ds(row0 + (N_CHUNKS - 1) * CHUNK, CHUNK), pl.ds(0, D)],
        semw[1],
    ).wait()
    pltpu.sync_copy(accv, part_hbm.at[wid])


_PB = 16384
_PG = -(-VOCAB // _PB)


def _tc_pad_body(wt_ref, out_ref):
    x = wt_ref[...]
    y = jnp.transpose(x, (1, 0))
    out_ref[:, pl.ds(0, D)] = y


_tc_pad = pl.pallas_call(
    _tc_pad_body,
    grid=(_PG,),
    in_specs=[pl.BlockSpec((D, _PB), lambda i: (0, i))],
    out_specs=pl.BlockSpec((_PB, 128), lambda i: (i, 0)),
    out_shape=jax.ShapeDtypeStruct((VOCAB, 128), jnp.float32),
)


def kernel(inputs, targets, wte):
    idx2 = (inputs.astype(jnp.int32).reshape(-1) * 2).reshape(N // SUB, SUB)
    tgt = targets.astype(jnp.int32).reshape(N)
    table = _tc_pad(wte.T).reshape(2 * VOCAB, D)
    logits_pad, partials = _sc_embed_ce(idx2, tgt, table)
    logits2 = logits_pad[:, :D]
    loss = jnp.sum(partials) * (1.0 / N)
    return (logits2, loss)
